# baseline (speedup 1.0000x reference)
import functools

import jax
import jax.numpy as jnp
from jax import lax
from jax.experimental import pallas as pl
from jax.experimental.pallas import tpu as pltpu
from jax.experimental.pallas import tpu_sc as plsc

_BATCH = 16384
_DIM = 64
_NU = 100000
_NI = 1000000

_info = plsc.get_sparse_core_info()
_NC, _NS, _L = _info.num_cores, _info.num_subcores, _info.num_lanes
_NW = _NC * _NS
_BPW = _BATCH // _NW
_CHUNK = 128
_NCH = _BPW // _CHUNK
_GPC = _CHUNK // _L
_DBLK = 8


def _make_conv(n_rows):
    nwin = n_rows // 128
    tail = n_rows % 128
    kmax = -(-nwin // _NW)
    mesh = plsc.VectorSubcoreMesh(core_axis_name="c", subcore_axis_name="s")

    @functools.partial(
        pl.kernel,
        mesh=mesh,
        out_type=jax.ShapeDtypeStruct((n_rows // 2, 128), jnp.float32),
        compiler_params=pltpu.CompilerParams(
            use_tc_tiling_on_sc=True, needs_layout_passes=False),
        scratch_types=[
            pltpu.VMEM((2, 8, 8, 128), jnp.float32),
            pltpu.VMEM((2, _DIM, 128), jnp.float32),
            pltpu.SemaphoreType.DMA,
            pltpu.SemaphoreType.DMA,
            pltpu.SemaphoreType.DMA,
        ],
    )
    def conv(tab_t, out_hbm, inb, outb, isem0, isem1, osem):
        wid = lax.axis_index("s") * _NC + lax.axis_index("c")
        nvalid = (nwin - wid + _NW - 1) // _NW
        lanes = lax.iota(jnp.int32, _L)
        isems = [isem0, isem1]

        def transpose_rows(b, ngroups):
            def tgroup(g, carry):
                rows = lanes + g * _L
                prv = rows >> 1
                colb = (rows & 1) << 6
                for d in range(_DIM):
                    dh = jnp.full((_L,), 0, jnp.int32) + (d >> 3)
                    dl = jnp.full((_L,), 0, jnp.int32) + (d & 7)
                    v = plsc.load_gather(inb.at[b], [dh, dl, rows])
                    plsc.store_scatter(outb.at[b], [prv, colb + d], v)
                return carry

            lax.fori_loop(0, ngroups, tgroup, 0)

        def start_of(t):
            return pl.multiple_of(t * 128, 128)

        def fire_in(k, b):
            t = wid + _NW * k

            @pl.when(t < nwin)
            def _():
                s0 = start_of(t)
                for dh in range(8):
                    pltpu.async_copy(
                        tab_t.at[pl.ds(dh * 8, 8), pl.ds(s0, 128)],
                        inb.at[b, dh], isems[b])

        def sub(k, b):
            t = wid + _NW * k
            fire_in(k + 1, 1 - b)

            @pl.when(t < nwin)
            def _():
                for dh in range(8):
                    pltpu.make_async_copy(
                        tab_t.at[pl.ds(0, 8), pl.ds(0, 128)],
                        inb.at[b, dh], isems[b]).wait()

                @pl.when(k >= 2)
                def _():
                    pltpu.make_async_copy(
                        out_hbm.at[pl.ds(0, _DIM)], outb.at[b], osem).wait()

                transpose_rows(b, 128 // _L)
                pltpu.async_copy(
                    outb.at[b],
                    out_hbm.at[pl.ds(pl.multiple_of(t * _DIM, _DIM), _DIM)],
                    osem)

        fire_in(0, 0)

        def body(k2, carry):
            sub(2 * k2, 0)
            sub(2 * k2 + 1, 1)
            return carry

        lax.fori_loop(0, kmax // 2, body, 0)
        if kmax % 2:
            sub(kmax - 1, 0)

        @pl.when(nvalid >= 1)
        def _():
            pltpu.make_async_copy(
                out_hbm.at[pl.ds(0, _DIM)], outb.at[0], osem).wait()

        @pl.when(nvalid >= 2)
        def _():
            pltpu.make_async_copy(
                out_hbm.at[pl.ds(0, _DIM)], outb.at[1], osem).wait()

        if tail:
            @pl.when(wid == 0)
            def _():
                tstart = pl.multiple_of(wid * 0 + nwin * 128, 128)
                for dh in range(8):
                    pltpu.sync_copy(
                        tab_t.at[pl.ds(dh * 8, 8), pl.ds(tstart, 128)],
                        inb.at[0, dh])

                transpose_rows(0, tail // _L)
                pltpu.sync_copy(
                    outb.at[0, pl.ds(0, tail // 2)],
                    out_hbm.at[pl.ds(nwin * _DIM, tail // 2)])

    return conv


def _make_gather():
    mesh = plsc.VectorSubcoreMesh(core_axis_name="c", subcore_axis_name="s")

    @functools.partial(
        pl.kernel,
        mesh=mesh,
        out_type=jax.ShapeDtypeStruct((_BATCH,), jnp.float32),
        compiler_params=pltpu.CompilerParams(
            use_tc_tiling_on_sc=False, needs_layout_passes=False),
        scratch_types=[
            pltpu.VMEM((_BPW,), jnp.int32),
            pltpu.VMEM((_BPW,), jnp.int32),
            pltpu.VMEM((_BPW,), jnp.int32),
            pltpu.VMEM((_BPW,), jnp.int32),
            pltpu.VMEM((_BPW,), jnp.int32),
            pltpu.VMEM((_BPW,), jnp.int32),
            pltpu.VMEM((2, _CHUNK, 128), jnp.float32),
            pltpu.VMEM((2, _CHUNK, 128), jnp.float32),
            pltpu.VMEM((2, _CHUNK, 128), jnp.float32),
            pltpu.VMEM((_BPW,), jnp.float32),
            pltpu.SemaphoreType.DMA,
            pltpu.SemaphoreType.DMA,
        ],
    )
    def k(user_hbm, item1_hbm, item2_hbm, utab_hbm, itab_hbm, out_hbm,
          uidx, aidx, bidx, upk, apk, bpk, urow, arow, brow, outv,
          sem0, sem1):
        wid = lax.axis_index("s") * _NC + lax.axis_index("c")
        base = wid * _BPW
        pltpu.sync_copy(user_hbm.at[pl.ds(base, _BPW)], uidx)
        pltpu.sync_copy(item1_hbm.at[pl.ds(base, _BPW)], aidx)
        pltpu.sync_copy(item2_hbm.at[pl.ds(base, _BPW)], bidx)

        def pack(i, carry):
            s = pl.ds(i * _L, _L)
            upk[s] = uidx[s] >> 1
            apk[s] = aidx[s] >> 1
            bpk[s] = bidx[s] >> 1
            return carry

        lax.fori_loop(0, _BPW // _L, pack, 0)

        sems = [sem0, sem1]

        def fire(c):
            s = pl.ds(c * _CHUNK, _CHUNK)
            b = c % 2
            return [
                pltpu.async_copy(utab_hbm.at[upk.at[s]], urow.at[b], sems[b]),
                pltpu.async_copy(itab_hbm.at[apk.at[s]], arow.at[b], sems[b]),
                pltpu.async_copy(itab_hbm.at[bpk.at[s]], brow.at[b], sems[b]),
            ]

        lanes = lax.iota(jnp.int32, _L)
        pending = {0: fire(0)}

        for c in range(_NCH):
            if c + 1 < _NCH:
                pending[c + 1] = fire(c + 1)
            for cp in pending.pop(c):
                cp.wait()
            b = c % 2
            ur, ar, br = urow.at[b], arow.at[b], brow.at[b]

            def group(g, carry, c=c, ur=ur, ar=ar, br=br):
                goff = c * _CHUNK + g * _L
                s = pl.ds(goff, _L)
                rows = lanes + g * _L
                off_u = (uidx[s] & 1) << 6
                off_a = (aidx[s] & 1) << 6
                off_b = (bidx[s] & 1) << 6

                def dblk(t, acc, ur=ur, ar=ar, br=br, rows=rows,
                         off_u=off_u, off_a=off_a, off_b=off_b):
                    d0 = t * _DBLK
                    ps = []
                    for j in range(_DBLK):
                        d = d0 + j
                        u = plsc.load_gather(ur, [rows, off_u + d])
                        a = plsc.load_gather(ar, [rows, off_a + d])
                        bb = plsc.load_gather(br, [rows, off_b + d])
                        ps.append((u + a) * (u + bb))
                    s1 = (ps[0] + ps[1]) + (ps[2] + ps[3])
                    s2 = (ps[4] + ps[5]) + (ps[6] + ps[7])
                    return acc + (s1 + s2)

                dot = lax.fori_loop(
                    0, _DIM // _DBLK, dblk, jnp.zeros((_L,), jnp.float32))
                outv[pl.ds(goff, _L)] = 1.0 / (1.0 + jnp.exp(-dot))
                return carry

            lax.fori_loop(0, _GPC, group, 0)

        pltpu.sync_copy(outv, out_hbm.at[pl.ds(base, _BPW)])

    return k


_conv_user = _make_conv(_NU)
_conv_item = _make_conv(_NI)
_gather = _make_gather()


def kernel(user, item1, item2, user_table, item_table):
    utp = _conv_user(user_table.T)
    itp = _conv_item(item_table.T)
    return _gather(user, item1, item2, utp, itp)


# speedup vs baseline: 1.6742x; 1.6742x over previous
import functools

import jax
import jax.numpy as jnp
from jax import lax
from jax.experimental import pallas as pl
from jax.experimental.pallas import tpu as pltpu
from jax.experimental.pallas import tpu_sc as plsc

_BATCH = 16384
_DIM = 64
_NU = 100000
_NI = 1000000

_info = plsc.get_sparse_core_info()
_NC, _NS, _L = _info.num_cores, _info.num_subcores, _info.num_lanes
_NW = _NC * _NS
_BPW = _BATCH // _NW
_CHUNK = 128
_NCH = _BPW // _CHUNK
_GPC = _CHUNK // _L
_DBLK = 8


def _make_conv(n_rows):
    nwin = n_rows // 128
    tail = n_rows % 128
    kmax = -(-nwin // _NW)
    mesh = plsc.VectorSubcoreMesh(core_axis_name="c", subcore_axis_name="s")

    @functools.partial(
        pl.kernel,
        mesh=mesh,
        out_type=jax.ShapeDtypeStruct((n_rows // 2, 128), jnp.float32),
        compiler_params=pltpu.CompilerParams(
            use_tc_tiling_on_sc=True, needs_layout_passes=False),
        scratch_types=[
            pltpu.VMEM((2, 8, 8, 128), jnp.float32),
            pltpu.VMEM((2, _DIM, 128), jnp.float32),
            pltpu.SemaphoreType.DMA,
            pltpu.SemaphoreType.DMA,
            pltpu.SemaphoreType.DMA,
        ],
    )
    def conv(tab_t, out_hbm, inb, outb, isem0, isem1, osem):
        wid = lax.axis_index("s") * _NC + lax.axis_index("c")
        nvalid = (nwin - wid + _NW - 1) // _NW
        lanes = lax.iota(jnp.int32, _L)
        isems = [isem0, isem1]

        def transpose_rows(b, ngroups):
            def tgroup(g, carry):
                rows = lanes + g * _L
                prv = rows >> 1
                colb = (rows & 1) << 6
                for q in range(_DIM // _L):
                    for s in range(_L):
                        dvec = ((lanes + s) & (_L - 1)) + q * _L
                        v = plsc.load_gather(
                            inb.at[b], [dvec >> 3, dvec & 7, rows])
                        plsc.store_scatter(
                            outb.at[b], [prv, colb + dvec], v)
                return carry

            lax.fori_loop(0, ngroups, tgroup, 0)

        def start_of(t):
            return pl.multiple_of(t * 128, 128)

        def fire_in(k, b):
            t = wid + _NW * k

            @pl.when(t < nwin)
            def _():
                s0 = start_of(t)
                for dh in range(8):
                    pltpu.async_copy(
                        tab_t.at[pl.ds(dh * 8, 8), pl.ds(s0, 128)],
                        inb.at[b, dh], isems[b])

        def sub(k, b):
            t = wid + _NW * k
            fire_in(k + 1, 1 - b)

            @pl.when(t < nwin)
            def _():
                for dh in range(8):
                    pltpu.make_async_copy(
                        tab_t.at[pl.ds(0, 8), pl.ds(0, 128)],
                        inb.at[b, dh], isems[b]).wait()

                @pl.when(k >= 2)
                def _():
                    pltpu.make_async_copy(
                        out_hbm.at[pl.ds(0, _DIM)], outb.at[b], osem).wait()

                transpose_rows(b, 128 // _L)
                pltpu.async_copy(
                    outb.at[b],
                    out_hbm.at[pl.ds(pl.multiple_of(t * _DIM, _DIM), _DIM)],
                    osem)

        fire_in(0, 0)

        def body(k2, carry):
            sub(2 * k2, 0)
            sub(2 * k2 + 1, 1)
            return carry

        lax.fori_loop(0, kmax // 2, body, 0)
        if kmax % 2:
            sub(kmax - 1, 0)

        @pl.when(nvalid >= 1)
        def _():
            pltpu.make_async_copy(
                out_hbm.at[pl.ds(0, _DIM)], outb.at[0], osem).wait()

        @pl.when(nvalid >= 2)
        def _():
            pltpu.make_async_copy(
                out_hbm.at[pl.ds(0, _DIM)], outb.at[1], osem).wait()

        if tail:
            @pl.when(wid == 0)
            def _():
                tstart = pl.multiple_of(wid * 0 + nwin * 128, 128)
                for dh in range(8):
                    pltpu.sync_copy(
                        tab_t.at[pl.ds(dh * 8, 8), pl.ds(tstart, 128)],
                        inb.at[0, dh])

                transpose_rows(0, tail // _L)
                pltpu.sync_copy(
                    outb.at[0, pl.ds(0, tail // 2)],
                    out_hbm.at[pl.ds(nwin * _DIM, tail // 2)])

    return conv


def _make_gather():
    mesh = plsc.VectorSubcoreMesh(core_axis_name="c", subcore_axis_name="s")

    @functools.partial(
        pl.kernel,
        mesh=mesh,
        out_type=jax.ShapeDtypeStruct((_BATCH,), jnp.float32),
        compiler_params=pltpu.CompilerParams(
            use_tc_tiling_on_sc=False, needs_layout_passes=False),
        scratch_types=[
            pltpu.VMEM((_BPW,), jnp.int32),
            pltpu.VMEM((_BPW,), jnp.int32),
            pltpu.VMEM((_BPW,), jnp.int32),
            pltpu.VMEM((_BPW,), jnp.int32),
            pltpu.VMEM((_BPW,), jnp.int32),
            pltpu.VMEM((_BPW,), jnp.int32),
            pltpu.VMEM((2, _CHUNK, 128), jnp.float32),
            pltpu.VMEM((2, _CHUNK, 128), jnp.float32),
            pltpu.VMEM((2, _CHUNK, 128), jnp.float32),
            pltpu.VMEM((_BPW,), jnp.float32),
            pltpu.SemaphoreType.DMA,
            pltpu.SemaphoreType.DMA,
        ],
    )
    def k(user_hbm, item1_hbm, item2_hbm, utab_hbm, itab_hbm, out_hbm,
          uidx, aidx, bidx, upk, apk, bpk, urow, arow, brow, outv,
          sem0, sem1):
        wid = lax.axis_index("s") * _NC + lax.axis_index("c")
        base = wid * _BPW
        pltpu.sync_copy(user_hbm.at[pl.ds(base, _BPW)], uidx)
        pltpu.sync_copy(item1_hbm.at[pl.ds(base, _BPW)], aidx)
        pltpu.sync_copy(item2_hbm.at[pl.ds(base, _BPW)], bidx)

        def pack(i, carry):
            s = pl.ds(i * _L, _L)
            upk[s] = uidx[s] >> 1
            apk[s] = aidx[s] >> 1
            bpk[s] = bidx[s] >> 1
            return carry

        lax.fori_loop(0, _BPW // _L, pack, 0)

        sems = [sem0, sem1]

        def fire(c):
            s = pl.ds(c * _CHUNK, _CHUNK)
            b = c % 2
            return [
                pltpu.async_copy(utab_hbm.at[upk.at[s]], urow.at[b], sems[b]),
                pltpu.async_copy(itab_hbm.at[apk.at[s]], arow.at[b], sems[b]),
                pltpu.async_copy(itab_hbm.at[bpk.at[s]], brow.at[b], sems[b]),
            ]

        lanes = lax.iota(jnp.int32, _L)
        pending = {0: fire(0)}

        for c in range(_NCH):
            if c + 1 < _NCH:
                pending[c + 1] = fire(c + 1)
            for cp in pending.pop(c):
                cp.wait()
            b = c % 2
            ur, ar, br = urow.at[b], arow.at[b], brow.at[b]

            def group(g, carry, c=c, ur=ur, ar=ar, br=br):
                goff = c * _CHUNK + g * _L
                s = pl.ds(goff, _L)
                rows = lanes + g * _L
                off_u = (uidx[s] & 1) << 6
                off_a = (aidx[s] & 1) << 6
                off_b = (bidx[s] & 1) << 6

                def dblk(t, acc, ur=ur, ar=ar, br=br, rows=rows,
                         off_u=off_u, off_a=off_a, off_b=off_b):
                    ps = []
                    for s in range(_L):
                        dvec = ((lanes + s) & (_L - 1)) + t * _L
                        u = plsc.load_gather(ur, [rows, off_u + dvec])
                        a = plsc.load_gather(ar, [rows, off_a + dvec])
                        bb = plsc.load_gather(br, [rows, off_b + dvec])
                        ps.append((u + a) * (u + bb))
                    s1 = (ps[0] + ps[1]) + (ps[2] + ps[3])
                    s2 = (ps[4] + ps[5]) + (ps[6] + ps[7])
                    s3 = (ps[8] + ps[9]) + (ps[10] + ps[11])
                    s4 = (ps[12] + ps[13]) + (ps[14] + ps[15])
                    return acc + ((s1 + s2) + (s3 + s4))

                dot = lax.fori_loop(
                    0, _DIM // _L, dblk, jnp.zeros((_L,), jnp.float32))
                outv[pl.ds(goff, _L)] = 1.0 / (1.0 + jnp.exp(-dot))
                return carry

            lax.fori_loop(0, _GPC, group, 0)

        pltpu.sync_copy(outv, out_hbm.at[pl.ds(base, _BPW)])

    return k


_conv_user = _make_conv(_NU)
_conv_item = _make_conv(_NI)
_gather = _make_gather()


def kernel(user, item1, item2, user_table, item_table):
    utp = _conv_user(user_table.T)
    itp = _conv_item(item_table.T)
    return _gather(user, item1, item2, utp, itp)


# speedup vs baseline: 1.8951x; 1.1319x over previous
import functools

import jax
import jax.numpy as jnp
from jax import lax
from jax.experimental import pallas as pl
from jax.experimental.pallas import tpu as pltpu
from jax.experimental.pallas import tpu_sc as plsc

_BATCH = 16384
_DIM = 64
_NU = 100000
_NI = 1000000

_info = plsc.get_sparse_core_info()
_NC, _NS, _L = _info.num_cores, _info.num_subcores, _info.num_lanes
_NW = _NC * _NS
_BPW = _BATCH // _NW
_CHUNK = 128
_NCH = _BPW // _CHUNK
_GPC = _CHUNK // _L
_DBLK = 8


def _make_conv(n_rows):
    win = 256
    nwin = n_rows // win
    tail = n_rows % win
    kmax = -(-nwin // _NW)
    mesh = plsc.VectorSubcoreMesh(core_axis_name="c", subcore_axis_name="s")

    @functools.partial(
        pl.kernel,
        mesh=mesh,
        out_type=jax.ShapeDtypeStruct((n_rows // 2, 128), jnp.float32),
        compiler_params=pltpu.CompilerParams(
            use_tc_tiling_on_sc=True, needs_layout_passes=False),
        scratch_types=[
            pltpu.VMEM((2, 8, 8, 256), jnp.float32),
            pltpu.VMEM((2, 128, 128), jnp.float32),
            pltpu.SemaphoreType.DMA,
            pltpu.SemaphoreType.DMA,
            pltpu.SemaphoreType.DMA,
        ],
    )
    def conv(tab_t, out_hbm, inb, outb, isem0, isem1, osem):
        wid = lax.axis_index("s") * _NC + lax.axis_index("c")
        nvalid = (nwin - wid + _NW - 1) // _NW
        lanes = lax.iota(jnp.int32, _L)
        isems = [isem0, isem1]

        def transpose_rows(b, ngroups):
            def tgroup(g, carry):
                rows = lanes + g * _L
                prv = rows >> 1
                colb = (rows & 1) << 6
                for q in range(_DIM // _L):
                    for s in range(_L):
                        dvec = ((lanes + s) & (_L - 1)) + q * _L
                        v = plsc.load_gather(
                            inb.at[b], [dvec >> 3, dvec & 7, rows])
                        plsc.store_scatter(
                            outb.at[b], [prv, colb + dvec], v)
                return carry

            lax.fori_loop(0, ngroups, tgroup, 0)

        def start_of(t):
            return pl.multiple_of(t * win, 128)

        def fire_in(k, b):
            t = wid + _NW * k

            @pl.when(t < nwin)
            def _():
                s0 = start_of(t)
                for dh in range(8):
                    pltpu.async_copy(
                        tab_t.at[pl.ds(dh * 8, 8), pl.ds(s0, win)],
                        inb.at[b, dh], isems[b])

        def sub(k, b):
            t = wid + _NW * k
            fire_in(k + 1, 1 - b)

            @pl.when(t < nwin)
            def _():
                for dh in range(8):
                    pltpu.make_async_copy(
                        tab_t.at[pl.ds(0, 8), pl.ds(0, win)],
                        inb.at[b, dh], isems[b]).wait()

                @pl.when(k >= 2)
                def _():
                    pltpu.make_async_copy(
                        out_hbm.at[pl.ds(0, win // 2)], outb.at[b],
                        osem).wait()

                transpose_rows(b, win // _L)
                pltpu.async_copy(
                    outb.at[b],
                    out_hbm.at[pl.ds(pl.multiple_of(t * (win // 2), 8),
                                     win // 2)],
                    osem)

        fire_in(0, 0)

        def body(k2, carry):
            sub(2 * k2, 0)
            sub(2 * k2 + 1, 1)
            return carry

        lax.fori_loop(0, kmax // 2, body, 0)
        if kmax % 2:
            sub(kmax - 1, 0)

        @pl.when(nvalid >= 1)
        def _():
            pltpu.make_async_copy(
                out_hbm.at[pl.ds(0, win // 2)], outb.at[0], osem).wait()

        @pl.when(nvalid >= 2)
        def _():
            pltpu.make_async_copy(
                out_hbm.at[pl.ds(0, win // 2)], outb.at[1], osem).wait()

        if tail:
            @pl.when(wid == 0)
            def _():
                tstart = pl.multiple_of(wid * 0 + nwin * win, 128)
                tw = 128 * (-(-tail // 128))
                for dh in range(8):
                    pltpu.sync_copy(
                        tab_t.at[pl.ds(dh * 8, 8), pl.ds(tstart, tw)],
                        inb.at[0, dh, :, pl.ds(0, tw)])

                transpose_rows(0, tail // _L)
                pltpu.sync_copy(
                    outb.at[0, pl.ds(0, tail // 2)],
                    out_hbm.at[pl.ds(nwin * (win // 2), tail // 2)])

    return conv


def _make_gather():
    mesh = plsc.VectorSubcoreMesh(core_axis_name="c", subcore_axis_name="s")

    @functools.partial(
        pl.kernel,
        mesh=mesh,
        out_type=jax.ShapeDtypeStruct((_BATCH,), jnp.float32),
        compiler_params=pltpu.CompilerParams(
            use_tc_tiling_on_sc=False, needs_layout_passes=False),
        scratch_types=[
            pltpu.VMEM((_BPW,), jnp.int32),
            pltpu.VMEM((_BPW,), jnp.int32),
            pltpu.VMEM((_BPW,), jnp.int32),
            pltpu.VMEM((_BPW,), jnp.int32),
            pltpu.VMEM((_BPW,), jnp.int32),
            pltpu.VMEM((_BPW,), jnp.int32),
            pltpu.VMEM((2, _CHUNK, 128), jnp.float32),
            pltpu.VMEM((2, _CHUNK, 128), jnp.float32),
            pltpu.VMEM((2, _CHUNK, 128), jnp.float32),
            pltpu.VMEM((_BPW,), jnp.float32),
            pltpu.SemaphoreType.DMA,
            pltpu.SemaphoreType.DMA,
        ],
    )
    def k(user_hbm, item1_hbm, item2_hbm, utab_hbm, itab_hbm, out_hbm,
          uidx, aidx, bidx, upk, apk, bpk, urow, arow, brow, outv,
          sem0, sem1):
        wid = lax.axis_index("s") * _NC + lax.axis_index("c")
        base = wid * _BPW
        pltpu.sync_copy(user_hbm.at[pl.ds(base, _BPW)], uidx)
        pltpu.sync_copy(item1_hbm.at[pl.ds(base, _BPW)], aidx)
        pltpu.sync_copy(item2_hbm.at[pl.ds(base, _BPW)], bidx)

        def pack(i, carry):
            s = pl.ds(i * _L, _L)
            upk[s] = uidx[s] >> 1
            apk[s] = aidx[s] >> 1
            bpk[s] = bidx[s] >> 1
            return carry

        lax.fori_loop(0, _BPW // _L, pack, 0)

        sems = [sem0, sem1]

        def fire(c):
            s = pl.ds(c * _CHUNK, _CHUNK)
            b = c % 2
            return [
                pltpu.async_copy(utab_hbm.at[upk.at[s]], urow.at[b], sems[b]),
                pltpu.async_copy(itab_hbm.at[apk.at[s]], arow.at[b], sems[b]),
                pltpu.async_copy(itab_hbm.at[bpk.at[s]], brow.at[b], sems[b]),
            ]

        lanes = lax.iota(jnp.int32, _L)
        pending = {0: fire(0)}

        for c in range(_NCH):
            if c + 1 < _NCH:
                pending[c + 1] = fire(c + 1)
            for cp in pending.pop(c):
                cp.wait()
            b = c % 2
            ur, ar, br = urow.at[b], arow.at[b], brow.at[b]

            def group(g, carry, c=c, ur=ur, ar=ar, br=br):
                goff = c * _CHUNK + g * _L
                s = pl.ds(goff, _L)
                rows = lanes + g * _L
                off_u = (uidx[s] & 1) << 6
                off_a = (aidx[s] & 1) << 6
                off_b = (bidx[s] & 1) << 6

                def dblk(t, acc, ur=ur, ar=ar, br=br, rows=rows,
                         off_u=off_u, off_a=off_a, off_b=off_b):
                    ps = []
                    for s in range(_L):
                        dvec = ((lanes + s) & (_L - 1)) + t * _L
                        u = plsc.load_gather(ur, [rows, off_u + dvec])
                        a = plsc.load_gather(ar, [rows, off_a + dvec])
                        bb = plsc.load_gather(br, [rows, off_b + dvec])
                        ps.append((u + a) * (u + bb))
                    s1 = (ps[0] + ps[1]) + (ps[2] + ps[3])
                    s2 = (ps[4] + ps[5]) + (ps[6] + ps[7])
                    s3 = (ps[8] + ps[9]) + (ps[10] + ps[11])
                    s4 = (ps[12] + ps[13]) + (ps[14] + ps[15])
                    return acc + ((s1 + s2) + (s3 + s4))

                dot = lax.fori_loop(
                    0, _DIM // _L, dblk, jnp.zeros((_L,), jnp.float32))
                outv[pl.ds(goff, _L)] = 1.0 / (1.0 + jnp.exp(-dot))
                return carry

            lax.fori_loop(0, _GPC, group, 0)

        pltpu.sync_copy(outv, out_hbm.at[pl.ds(base, _BPW)])

    return k


_conv_user = _make_conv(_NU)
_conv_item = _make_conv(_NI)
_gather = _make_gather()


def kernel(user, item1, item2, user_table, item_table):
    utp = _conv_user(user_table.T)
    itp = _conv_item(item_table.T)
    return _gather(user, item1, item2, utp, itp)


# speedup vs baseline: 2.4030x; 1.2680x over previous
import functools

import jax
import jax.numpy as jnp
from jax import lax
from jax.experimental import pallas as pl
from jax.experimental.pallas import tpu as pltpu
from jax.experimental.pallas import tpu_sc as plsc

_BATCH = 16384
_DIM = 64

_info = plsc.get_sparse_core_info()
_NC, _NS, _L = _info.num_cores, _info.num_subcores, _info.num_lanes
_NW = _NC * _NS
_BPW = _BATCH // _NW
_CHUNK = 128
_NCH = _BPW // _CHUNK
_GPC = _CHUNK // _L


def _make_sc_kernel():
    mesh = plsc.VectorSubcoreMesh(core_axis_name="c", subcore_axis_name="s")

    @functools.partial(
        pl.kernel,
        mesh=mesh,
        out_type=jax.ShapeDtypeStruct((_BATCH,), jnp.float32),
        compiler_params=pltpu.CompilerParams(
            use_tc_tiling_on_sc=False, needs_layout_passes=False),
        scratch_types=[
            pltpu.VMEM((_BPW,), jnp.int32),
            pltpu.VMEM((_BPW,), jnp.int32),
            pltpu.VMEM((_BPW,), jnp.int32),
            pltpu.VMEM((2, _CHUNK, _DIM), jnp.float32),
            pltpu.VMEM((2, _CHUNK, _DIM), jnp.float32),
            pltpu.VMEM((2, _CHUNK, _DIM), jnp.float32),
            pltpu.VMEM((_BPW,), jnp.float32),
            pltpu.SemaphoreType.DMA,
            pltpu.SemaphoreType.DMA,
        ],
    )
    def k(user_hbm, item1_hbm, item2_hbm, utab_hbm, itab_hbm, out_hbm,
          uidx, aidx, bidx, urow, arow, brow, outv, sem0, sem1):
        wid = lax.axis_index("s") * _NC + lax.axis_index("c")
        base = wid * _BPW
        pltpu.sync_copy(user_hbm.at[pl.ds(base, _BPW)], uidx)
        pltpu.sync_copy(item1_hbm.at[pl.ds(base, _BPW)], aidx)
        pltpu.sync_copy(item2_hbm.at[pl.ds(base, _BPW)], bidx)

        sems = [sem0, sem1]

        def fire(c):
            s = pl.ds(c * _CHUNK, _CHUNK)
            b = c % 2
            return [
                pltpu.async_copy(utab_hbm.at[uidx.at[s]], urow.at[b], sems[b]),
                pltpu.async_copy(itab_hbm.at[aidx.at[s]], arow.at[b], sems[b]),
                pltpu.async_copy(itab_hbm.at[bidx.at[s]], brow.at[b], sems[b]),
            ]

        lanes = lax.iota(jnp.int32, _L)
        pending = {0: fire(0)}

        for c in range(_NCH):
            if c + 1 < _NCH:
                pending[c + 1] = fire(c + 1)
            for cp in pending.pop(c):
                cp.wait()
            b = c % 2
            ur, ar, br = urow.at[b], arow.at[b], brow.at[b]

            def group(g, carry, c=c, ur=ur, ar=ar, br=br):
                goff = c * _CHUNK + g * _L
                rows = lanes + g * _L

                def dblk(t, acc, ur=ur, ar=ar, br=br, rows=rows):
                    ps = []
                    for s in range(_L):
                        dvec = ((lanes + s) & (_L - 1)) + t * _L
                        u = plsc.load_gather(ur, [rows, dvec])
                        a = plsc.load_gather(ar, [rows, dvec])
                        bb = plsc.load_gather(br, [rows, dvec])
                        ps.append((u + a) * (u + bb))
                    s1 = (ps[0] + ps[1]) + (ps[2] + ps[3])
                    s2 = (ps[4] + ps[5]) + (ps[6] + ps[7])
                    s3 = (ps[8] + ps[9]) + (ps[10] + ps[11])
                    s4 = (ps[12] + ps[13]) + (ps[14] + ps[15])
                    return acc + ((s1 + s2) + (s3 + s4))

                dot = lax.fori_loop(
                    0, _DIM // _L, dblk, jnp.zeros((_L,), jnp.float32))
                outv[pl.ds(goff, _L)] = 1.0 / (1.0 + jnp.exp(-dot))
                return carry

            lax.fori_loop(0, _GPC, group, 0)

        pltpu.sync_copy(outv, out_hbm.at[pl.ds(base, _BPW)])

    return k


_sc_kernel = _make_sc_kernel()


def kernel(user, item1, item2, user_table, item_table):
    return _sc_kernel(user, item1, item2, user_table, item_table)
